# Initial kernel scaffold; baseline (speedup 1.0000x reference)
import jax
import jax.numpy as jnp
from jax.experimental import pallas as pl


def kernel(x, edge_index, Wl, bl, Wr, br, att, bias):
    raise NotImplementedError("write your pallas kernel here")


# speedup vs baseline: 12.0154x; 12.0154x over previous
import functools

import jax
import jax.numpy as jnp
from jax import lax
from jax.experimental import pallas as pl
from jax.experimental.pallas import tpu as pltpu
from jax.experimental.pallas import tpu_sc as plsc

H = 4
C = 32
D = H * C
NC = 2
NS = 16
NW = NC * NS
CH = 80
L = 16
NP = 10240
DR = NP * H // D


def _project_body(x_ref, wl_ref, bl_ref, wr_ref, br_ref, xl_ref, xr_ref):
    xb = x_ref[...]
    xl_ref[...] = jnp.dot(xb, wl_ref[...], preferred_element_type=jnp.float32) + bl_ref[...]
    xr_ref[...] = jnp.dot(xb, wr_ref[...], preferred_element_type=jnp.float32) + br_ref[...]


def _tc_project(x, Wl, bl, Wr, br):
    n = x.shape[0]
    B = 512
    grid = n // B
    return pl.pallas_call(
        _project_body,
        grid=(grid,),
        in_specs=[
            pl.BlockSpec((B, D), lambda i: (i, 0)),
            pl.BlockSpec((D, D), lambda i: (0, 0)),
            pl.BlockSpec((1, D), lambda i: (0, 0)),
            pl.BlockSpec((D, D), lambda i: (0, 0)),
            pl.BlockSpec((1, D), lambda i: (0, 0)),
        ],
        out_specs=[
            pl.BlockSpec((B, D), lambda i: (i, 0)),
            pl.BlockSpec((B, D), lambda i: (i, 0)),
        ],
        out_shape=[
            jax.ShapeDtypeStruct((n, D), jnp.float32),
            jax.ShapeDtypeStruct((n, D), jnp.float32),
        ],
    )(x, Wl, bl.reshape(1, D), Wr, br.reshape(1, D))


def _sc_edge_pass(src, dst, xl, xr, att):
    e = src.shape[0]
    ept = e // NW
    nchunk = ept // CH
    npt = NP // NS
    nz = 40
    dpt = DR // 8

    mesh = plsc.VectorSubcoreMesh(
        core_axis_name="c", subcore_axis_name="s", num_cores=NC, num_subcores=NS)

    @functools.partial(
        pl.kernel,
        out_type=jax.ShapeDtypeStruct((NC * (NP + DR), D), jnp.float32),
        mesh=mesh,
        scratch_types=[
            pltpu.VMEM((CH,), jnp.int32),
            pltpu.VMEM((CH,), jnp.int32),
            pltpu.VMEM((CH,), jnp.int32),
            pltpu.VMEM((CH, D), jnp.float32),
            pltpu.VMEM((CH, D), jnp.float32),
            pltpu.VMEM((CH, D), jnp.float32),
            pltpu.VMEM((nz, D), jnp.float32),
            pltpu.VMEM((D,), jnp.float32),
            pltpu.VMEM_SHARED((NP + DR, D), jnp.float32),
            pltpu.SemaphoreType.DMA,
            pltpu.SemaphoreType.DMA,
        ],
        compiler_params=pltpu.CompilerParams(needs_layout_passes=False),
    )
    def edge_kernel(src_hbm, dst_hbm, xl_hbm, xr_hbm, att_hbm, out_hbm,
                    src_v, dst_v, didx, xl_rows, xr_rows, dbuf, zbuf, att_v,
                    acc_sh, sem1, sem2):
        cid = lax.axis_index("c")
        sid = lax.axis_index("s")

        pltpu.sync_copy(att_hbm, att_v)

        zero16 = jnp.zeros((L,), jnp.float32)
        lane = lax.iota(jnp.int32, L)

        def zrow(i, _):
            def zcol(j, _):
                zbuf[i, pl.ds(j * L, L)] = zero16
                return 0
            lax.fori_loop(0, D // L, zcol, 0)
            return 0
        lax.fori_loop(0, nz, zrow, 0)

        rbase = sid * npt
        for k in range(npt // nz):
            pltpu.sync_copy(zbuf, acc_sh.at[pl.ds(rbase + k * nz, nz)])

        @pl.when(sid < 8)
        def _():
            pltpu.sync_copy(zbuf, acc_sh.at[pl.ds(NP + sid * dpt, dpt)])

        def zdrow(i, _):
            def zdcol(j, _):
                dbuf[i, pl.ds(j * L, L)] = zero16
                return 0
            lax.fori_loop(0, D // L, zdcol, 0)
            return 0
        lax.fori_loop(0, CH, zdrow, 0)

        plsc.subcore_barrier()

        ebase = (cid * NS + sid) * ept

        def chunk(i, _):
            off = ebase + i * CH
            pltpu.sync_copy(src_hbm.at[pl.ds(off, CH)], src_v)
            pltpu.sync_copy(dst_hbm.at[pl.ds(off, CH)], dst_v)
            pltpu.async_copy(xl_hbm.at[src_v], xl_rows, sem1).wait()
            pltpu.async_copy(xr_hbm.at[dst_v], xr_rows, sem2).wait()

            dcols = []
            for g in range(CH // L):
                eids = g * L + lane
                dv = plsc.load_gather(dst_v, [eids])
                didx[pl.ds(g * L, L)] = lax.shift_right_logical(dv, 5) + NP
                for h in range(H):
                    def accum(c, acc):
                        f = jnp.full((L,), h * C + c, jnp.int32)
                        m = (plsc.load_gather(xl_rows, [eids, f])
                             + plsc.load_gather(xr_rows, [eids, f]))
                        lr = jnp.maximum(m, 0.2 * m)
                        coef = plsc.load_gather(att_v, [f])
                        return acc + coef * lr
                    ev = lax.fori_loop(0, C, accum, jnp.zeros((L,), jnp.float32))
                    w = jnp.exp(ev)
                    col = jnp.bitwise_and(dv * 4 + h, 127)
                    plsc.store_scatter(dbuf, [eids, col], w)
                    dcols.append((eids, col))

                    def scale(c, _):
                        f = jnp.full((L,), h * C + c, jnp.int32)
                        a = plsc.load_gather(xl_rows, [eids, f])
                        plsc.store_scatter(xl_rows, [eids, f], w * a)
                        return 0
                    lax.fori_loop(0, C, scale, 0)

            pltpu.sync_copy(xl_rows, acc_sh.at[dst_v], add=True)
            pltpu.sync_copy(dbuf, acc_sh.at[didx], add=True)
            for eids, col in dcols:
                plsc.store_scatter(dbuf, [eids, col], jnp.zeros((L,), jnp.float32))
            return 0

        lax.fori_loop(0, nchunk, chunk, 0)

        plsc.subcore_barrier()

        obase = cid * (NP + DR) + sid * npt
        for k in range(npt // nz):
            pltpu.sync_copy(acc_sh.at[pl.ds(rbase + k * nz, nz)],
                            out_hbm.at[pl.ds(obase + k * nz, nz)])

        @pl.when(sid < 8)
        def _():
            dbase = cid * (NP + DR) + NP + sid * dpt
            pltpu.sync_copy(acc_sh.at[pl.ds(NP + sid * dpt, dpt)],
                            out_hbm.at[pl.ds(dbase, dpt)])

    return edge_kernel(src, dst, xl, xr, att)


def _finalize_body(p0_ref, p1_ref, d0_ref, d1_ref, xl_ref, xr_ref,
                   shead_ref, m_ref, bias_ref, o_ref):
    acc = p0_ref[...] + p1_ref[...]
    den = jnp.dot(d0_ref[...] + d1_ref[...], shead_ref[...],
                  preferred_element_type=jnp.float32)
    xlb = xl_ref[...]
    m = xlb + xr_ref[...]
    lr = jnp.maximum(m, 0.2 * m)
    w = jnp.exp(jnp.dot(lr, m_ref[...], preferred_element_type=jnp.float32))
    o_ref[...] = (acc + w * xlb) / (den + w + 1e-16) + bias_ref[...]


def _tc_finalize(p0, p1, d0, d1, xl, xr, shead, mblk, bias):
    B = 128
    grid = NP // B
    return pl.pallas_call(
        _finalize_body,
        grid=(grid,),
        in_specs=[
            pl.BlockSpec((B, D), lambda i: (i, 0)),
            pl.BlockSpec((B, D), lambda i: (i, 0)),
            pl.BlockSpec((B, H), lambda i: (i, 0)),
            pl.BlockSpec((B, H), lambda i: (i, 0)),
            pl.BlockSpec((B, D), lambda i: (i, 0)),
            pl.BlockSpec((B, D), lambda i: (i, 0)),
            pl.BlockSpec((H, D), lambda i: (0, 0)),
            pl.BlockSpec((D, D), lambda i: (0, 0)),
            pl.BlockSpec((1, D), lambda i: (0, 0)),
        ],
        out_specs=pl.BlockSpec((B, D), lambda i: (i, 0)),
        out_shape=jax.ShapeDtypeStruct((NP, D), jnp.float32),
    )(p0, p1, d0, d1, xl, xr, shead, mblk, bias.reshape(1, D))


def kernel(x, edge_index, Wl, bl, Wr, br, att, bias):
    n = x.shape[0]
    x_pad = jnp.pad(x, ((0, NP - n), (0, 0)))

    xl, xr = _tc_project(x_pad, Wl, bl, Wr, br)

    partial = _sc_edge_pass(edge_index[0], edge_index[1], xl, xr, att.reshape(D))

    p0 = partial[0:NP]
    d0 = partial[NP:NP + DR].reshape(NP, H)
    p1 = partial[NP + DR:2 * NP + DR]
    d1 = partial[2 * NP + DR:].reshape(NP, H)

    f = jnp.arange(D)
    shead = jnp.zeros((H, D), jnp.float32).at[f // C, f].set(1.0)
    head_eq = (f[:, None] // C) == (f[None, :] // C)
    mblk = jnp.where(head_eq, att.reshape(D)[:, None], 0.0).astype(jnp.float32)

    out = _tc_finalize(p0, p1, d0, d1, xl, xr, shead, mblk, bias)
    return out[:n]


# speedup vs baseline: 12.0817x; 1.0055x over previous
import functools

import jax
import jax.numpy as jnp
from jax import lax
from jax.experimental import pallas as pl
from jax.experimental.pallas import tpu as pltpu
from jax.experimental.pallas import tpu_sc as plsc

H = 4
C = 32
D = H * C
NC = 2
NS = 16
NW = NC * NS
CH = 80
L = 16
NP = 10240
DR = NP * H // D


def _project_body(x_ref, wl_ref, bl_ref, wr_ref, br_ref, xl_ref, xr_ref):
    xb = x_ref[...]
    xl_ref[...] = jnp.dot(xb, wl_ref[...], preferred_element_type=jnp.float32) + bl_ref[...]
    xr_ref[...] = jnp.dot(xb, wr_ref[...], preferred_element_type=jnp.float32) + br_ref[...]


def _tc_project(x, Wl, bl, Wr, br):
    n = x.shape[0]
    B = 512
    grid = n // B
    return pl.pallas_call(
        _project_body,
        grid=(grid,),
        in_specs=[
            pl.BlockSpec((B, D), lambda i: (i, 0)),
            pl.BlockSpec((D, D), lambda i: (0, 0)),
            pl.BlockSpec((1, D), lambda i: (0, 0)),
            pl.BlockSpec((D, D), lambda i: (0, 0)),
            pl.BlockSpec((1, D), lambda i: (0, 0)),
        ],
        out_specs=[
            pl.BlockSpec((B, D), lambda i: (i, 0)),
            pl.BlockSpec((B, D), lambda i: (i, 0)),
        ],
        out_shape=[
            jax.ShapeDtypeStruct((n, D), jnp.float32),
            jax.ShapeDtypeStruct((n, D), jnp.float32),
        ],
    )(x, Wl, bl.reshape(1, D), Wr, br.reshape(1, D))


def _sc_edge_pass(src, dst, xl, xr, att):
    e = src.shape[0]
    ept = e // NW
    nchunk = ept // CH
    npt = NP // NS
    nz = 40
    dpt = DR // 8

    mesh = plsc.VectorSubcoreMesh(
        core_axis_name="c", subcore_axis_name="s", num_cores=NC, num_subcores=NS)

    @functools.partial(
        pl.kernel,
        out_type=jax.ShapeDtypeStruct((NC * (NP + DR), D), jnp.float32),
        mesh=mesh,
        scratch_types=[
            pltpu.VMEM((CH,), jnp.int32),
            pltpu.VMEM((CH,), jnp.int32),
            pltpu.VMEM((CH,), jnp.int32),
            pltpu.VMEM((CH, D), jnp.float32),
            pltpu.VMEM((CH, D), jnp.float32),
            pltpu.VMEM((CH, D), jnp.float32),
            pltpu.VMEM((nz, D), jnp.float32),
            pltpu.VMEM((D,), jnp.float32),
            pltpu.VMEM_SHARED((NP + DR, D), jnp.float32),
            pltpu.SemaphoreType.DMA,
            pltpu.SemaphoreType.DMA,
        ],
        compiler_params=pltpu.CompilerParams(needs_layout_passes=False),
    )
    def edge_kernel(src_hbm, dst_hbm, xl_hbm, xr_hbm, att_hbm, out_hbm,
                    src_v, dst_v, didx, xl_rows, xr_rows, dbuf, zbuf, att_v,
                    acc_sh, sem1, sem2):
        cid = lax.axis_index("c")
        sid = lax.axis_index("s")

        pltpu.sync_copy(att_hbm, att_v)

        zero16 = jnp.zeros((L,), jnp.float32)
        lane = lax.iota(jnp.int32, L)

        def zrow(i, _):
            def zcol(j, _):
                zbuf[i, pl.ds(j * L, L)] = zero16
                return 0
            lax.fori_loop(0, D // L, zcol, 0)
            return 0
        lax.fori_loop(0, nz, zrow, 0)

        rbase = sid * npt
        for k in range(npt // nz):
            pltpu.sync_copy(zbuf, acc_sh.at[pl.ds(rbase + k * nz, nz)])

        @pl.when(sid < 8)
        def _():
            pltpu.sync_copy(zbuf, acc_sh.at[pl.ds(NP + sid * dpt, dpt)])

        def zdrow(i, _):
            def zdcol(j, _):
                dbuf[i, pl.ds(j * L, L)] = zero16
                return 0
            lax.fori_loop(0, D // L, zdcol, 0)
            return 0
        lax.fori_loop(0, CH, zdrow, 0)

        plsc.subcore_barrier()

        ebase = (cid * NS + sid) * ept

        def chunk(i, _):
            off = ebase + i * CH
            pltpu.sync_copy(src_hbm.at[pl.ds(off, CH)], src_v)
            pltpu.sync_copy(dst_hbm.at[pl.ds(off, CH)], dst_v)
            pltpu.async_copy(xl_hbm.at[src_v], xl_rows, sem1).wait()
            pltpu.async_copy(xr_hbm.at[dst_v], xr_rows, sem2).wait()

            dcols = []
            for g in range(CH // L):
                eids = g * L + lane
                dv = plsc.load_gather(dst_v, [eids])
                didx[pl.ds(g * L, L)] = lax.shift_right_logical(dv, 5) + NP
                for h in range(H):
                    U = 8

                    def accum(cb, acc):
                        c0 = h * C + cb * U
                        for u in range(U):
                            f = jnp.full((L,), c0 + u, jnp.int32)
                            m = (plsc.load_gather(xl_rows, [eids, f])
                                 + plsc.load_gather(xr_rows, [eids, f]))
                            lr = jnp.maximum(m, 0.2 * m)
                            coef = plsc.load_gather(att_v, [f])
                            acc = acc + coef * lr
                        return acc
                    ev = lax.fori_loop(0, C // U, accum,
                                       jnp.zeros((L,), jnp.float32))
                    w = jnp.exp(ev)
                    col = jnp.bitwise_and(dv * 4 + h, 127)
                    plsc.store_scatter(dbuf, [eids, col], w)
                    dcols.append((eids, col))

                    def scale(cb, _):
                        c0 = h * C + cb * U
                        for u in range(U):
                            f = jnp.full((L,), c0 + u, jnp.int32)
                            a = plsc.load_gather(xl_rows, [eids, f])
                            plsc.store_scatter(xl_rows, [eids, f], w * a)
                        return 0
                    lax.fori_loop(0, C // U, scale, 0)

            pltpu.sync_copy(xl_rows, acc_sh.at[dst_v], add=True)
            pltpu.sync_copy(dbuf, acc_sh.at[didx], add=True)
            for eids, col in dcols:
                plsc.store_scatter(dbuf, [eids, col], jnp.zeros((L,), jnp.float32))
            return 0

        lax.fori_loop(0, nchunk, chunk, 0)

        plsc.subcore_barrier()

        obase = cid * (NP + DR) + sid * npt
        for k in range(npt // nz):
            pltpu.sync_copy(acc_sh.at[pl.ds(rbase + k * nz, nz)],
                            out_hbm.at[pl.ds(obase + k * nz, nz)])

        @pl.when(sid < 8)
        def _():
            dbase = cid * (NP + DR) + NP + sid * dpt
            pltpu.sync_copy(acc_sh.at[pl.ds(NP + sid * dpt, dpt)],
                            out_hbm.at[pl.ds(dbase, dpt)])

    return edge_kernel(src, dst, xl, xr, att)


def _finalize_body(p0_ref, p1_ref, d0_ref, d1_ref, xl_ref, xr_ref,
                   shead_ref, m_ref, bias_ref, o_ref):
    acc = p0_ref[...] + p1_ref[...]
    den = jnp.dot(d0_ref[...] + d1_ref[...], shead_ref[...],
                  preferred_element_type=jnp.float32)
    xlb = xl_ref[...]
    m = xlb + xr_ref[...]
    lr = jnp.maximum(m, 0.2 * m)
    w = jnp.exp(jnp.dot(lr, m_ref[...], preferred_element_type=jnp.float32))
    o_ref[...] = (acc + w * xlb) / (den + w + 1e-16) + bias_ref[...]


def _tc_finalize(p0, p1, d0, d1, xl, xr, shead, mblk, bias):
    B = 128
    grid = NP // B
    return pl.pallas_call(
        _finalize_body,
        grid=(grid,),
        in_specs=[
            pl.BlockSpec((B, D), lambda i: (i, 0)),
            pl.BlockSpec((B, D), lambda i: (i, 0)),
            pl.BlockSpec((B, H), lambda i: (i, 0)),
            pl.BlockSpec((B, H), lambda i: (i, 0)),
            pl.BlockSpec((B, D), lambda i: (i, 0)),
            pl.BlockSpec((B, D), lambda i: (i, 0)),
            pl.BlockSpec((H, D), lambda i: (0, 0)),
            pl.BlockSpec((D, D), lambda i: (0, 0)),
            pl.BlockSpec((1, D), lambda i: (0, 0)),
        ],
        out_specs=pl.BlockSpec((B, D), lambda i: (i, 0)),
        out_shape=jax.ShapeDtypeStruct((NP, D), jnp.float32),
    )(p0, p1, d0, d1, xl, xr, shead, mblk, bias.reshape(1, D))


def kernel(x, edge_index, Wl, bl, Wr, br, att, bias):
    n = x.shape[0]
    x_pad = jnp.pad(x, ((0, NP - n), (0, 0)))

    xl, xr = _tc_project(x_pad, Wl, bl, Wr, br)

    partial = _sc_edge_pass(edge_index[0], edge_index[1], xl, xr, att.reshape(D))

    p0 = partial[0:NP]
    d0 = partial[NP:NP + DR].reshape(NP, H)
    p1 = partial[NP + DR:2 * NP + DR]
    d1 = partial[2 * NP + DR:].reshape(NP, H)

    f = jnp.arange(D)
    shead = jnp.zeros((H, D), jnp.float32).at[f // C, f].set(1.0)
    head_eq = (f[:, None] // C) == (f[None, :] // C)
    mblk = jnp.where(head_eq, att.reshape(D)[:, None], 0.0).astype(jnp.float32)

    out = _tc_finalize(p0, p1, d0, d1, xl, xr, shead, mblk, bias)
    return out[:n]


# speedup vs baseline: 12.4156x; 1.0276x over previous
import functools

import jax
import jax.numpy as jnp
from jax import lax
from jax.experimental import pallas as pl
from jax.experimental.pallas import tpu as pltpu
from jax.experimental.pallas import tpu_sc as plsc

H = 4
C = 32
D = H * C
NC = 2
NS = 16
NW = NC * NS
CH = 80
L = 16
NP = 10240
DR = NP * H // D


def _project_body(x_ref, wl_ref, bl_ref, wr_ref, br_ref, xl_ref, xr_ref):
    xb = x_ref[...]
    xl_ref[...] = jnp.dot(xb, wl_ref[...], preferred_element_type=jnp.float32) + bl_ref[...]
    xr_ref[...] = jnp.dot(xb, wr_ref[...], preferred_element_type=jnp.float32) + br_ref[...]


def _tc_project(x, Wl, bl, Wr, br):
    n = x.shape[0]
    B = 512
    grid = n // B
    return pl.pallas_call(
        _project_body,
        grid=(grid,),
        in_specs=[
            pl.BlockSpec((B, D), lambda i: (i, 0)),
            pl.BlockSpec((D, D), lambda i: (0, 0)),
            pl.BlockSpec((1, D), lambda i: (0, 0)),
            pl.BlockSpec((D, D), lambda i: (0, 0)),
            pl.BlockSpec((1, D), lambda i: (0, 0)),
        ],
        out_specs=[
            pl.BlockSpec((B, D), lambda i: (i, 0)),
            pl.BlockSpec((B, D), lambda i: (i, 0)),
        ],
        out_shape=[
            jax.ShapeDtypeStruct((n, D), jnp.float32),
            jax.ShapeDtypeStruct((n, D), jnp.float32),
        ],
    )(x, Wl, bl.reshape(1, D), Wr, br.reshape(1, D))


def _sc_edge_pass(src, dst, xl, xr, att):
    e = src.shape[0]
    ept = e // NW
    nchunk = ept // CH
    npt = NP // NS
    nz = 40
    dpt = DR // 8

    mesh = plsc.VectorSubcoreMesh(
        core_axis_name="c", subcore_axis_name="s", num_cores=NC, num_subcores=NS)

    @functools.partial(
        pl.kernel,
        out_type=jax.ShapeDtypeStruct((NC * (NP + DR), D), jnp.float32),
        mesh=mesh,
        scratch_types=[
            pltpu.VMEM((CH,), jnp.int32),
            pltpu.VMEM((CH,), jnp.int32),
            pltpu.VMEM((CH,), jnp.int32),
            pltpu.VMEM((CH, D), jnp.float32),
            pltpu.VMEM((CH, D), jnp.float32),
            pltpu.VMEM((CH, D), jnp.float32),
            pltpu.VMEM((nz, D), jnp.float32),
            pltpu.VMEM((D,), jnp.float32),
            pltpu.VMEM_SHARED((NP + DR, D), jnp.float32),
            pltpu.SemaphoreType.DMA,
            pltpu.SemaphoreType.DMA,
        ],
        compiler_params=pltpu.CompilerParams(needs_layout_passes=False),
    )
    def edge_kernel(src_hbm, dst_hbm, xl_hbm, xr_hbm, att_hbm, out_hbm,
                    src_v, dst_v, didx, xl_rows, xr_rows, dbuf, zbuf, att_v,
                    acc_sh, sem1, sem2):
        cid = lax.axis_index("c")
        sid = lax.axis_index("s")

        pltpu.sync_copy(att_hbm, att_v)

        zero16 = jnp.zeros((L,), jnp.float32)
        lane = lax.iota(jnp.int32, L)

        def zrow(i, _):
            def zcol(j, _):
                zbuf[i, pl.ds(j * L, L)] = zero16
                return 0
            lax.fori_loop(0, D // L, zcol, 0)
            return 0
        lax.fori_loop(0, nz, zrow, 0)

        rbase = sid * npt
        for k in range(npt // nz):
            pltpu.sync_copy(zbuf, acc_sh.at[pl.ds(rbase + k * nz, nz)])

        @pl.when(sid < 8)
        def _():
            pltpu.sync_copy(zbuf, acc_sh.at[pl.ds(NP + sid * dpt, dpt)])

        def zdrow(i, _):
            def zdcol(j, _):
                dbuf[i, pl.ds(j * L, L)] = zero16
                return 0
            lax.fori_loop(0, D // L, zdcol, 0)
            return 0
        lax.fori_loop(0, CH, zdrow, 0)

        plsc.subcore_barrier()

        ebase = (cid * NS + sid) * ept

        def chunk(i, _):
            off = ebase + i * CH
            pltpu.sync_copy(src_hbm.at[pl.ds(off, CH)], src_v)
            pltpu.sync_copy(dst_hbm.at[pl.ds(off, CH)], dst_v)
            cpl = pltpu.async_copy(xl_hbm.at[src_v], xl_rows, sem1)
            cpr = pltpu.async_copy(xr_hbm.at[dst_v], xr_rows, sem2)
            cpl.wait()
            cpr.wait()

            dcols = []
            for g in range(CH // L):
                eids = g * L + lane
                dv = plsc.load_gather(dst_v, [eids])
                didx[pl.ds(g * L, L)] = lax.shift_right_logical(dv, 5) + NP
                for h in range(H):
                    U = 8

                    def accum(cb, acc):
                        c0 = h * C + cb * U
                        for u in range(U):
                            f = jnp.full((L,), c0 + u, jnp.int32)
                            m = (plsc.load_gather(xl_rows, [eids, f])
                                 + plsc.load_gather(xr_rows, [eids, f]))
                            lr = jnp.maximum(m, 0.2 * m)
                            coef = plsc.load_gather(att_v, [f])
                            acc = acc + coef * lr
                        return acc
                    ev = lax.fori_loop(0, C // U, accum,
                                       jnp.zeros((L,), jnp.float32))
                    w = jnp.exp(ev)
                    col = jnp.bitwise_and(dv * 4 + h, 127)
                    plsc.store_scatter(dbuf, [eids, col], w)
                    dcols.append((eids, col))

                    def scale(cb, _):
                        c0 = h * C + cb * U
                        for u in range(U):
                            f = jnp.full((L,), c0 + u, jnp.int32)
                            a = plsc.load_gather(xl_rows, [eids, f])
                            plsc.store_scatter(xl_rows, [eids, f], w * a)
                        return 0
                    lax.fori_loop(0, C // U, scale, 0)

            pltpu.sync_copy(xl_rows, acc_sh.at[dst_v], add=True)
            pltpu.sync_copy(dbuf, acc_sh.at[didx], add=True)
            for eids, col in dcols:
                plsc.store_scatter(dbuf, [eids, col], jnp.zeros((L,), jnp.float32))
            return 0

        lax.fori_loop(0, nchunk, chunk, 0)

        plsc.subcore_barrier()

        obase = cid * (NP + DR) + sid * npt
        for k in range(npt // nz):
            pltpu.sync_copy(acc_sh.at[pl.ds(rbase + k * nz, nz)],
                            out_hbm.at[pl.ds(obase + k * nz, nz)])

        @pl.when(sid < 8)
        def _():
            dbase = cid * (NP + DR) + NP + sid * dpt
            pltpu.sync_copy(acc_sh.at[pl.ds(NP + sid * dpt, dpt)],
                            out_hbm.at[pl.ds(dbase, dpt)])

    return edge_kernel(src, dst, xl, xr, att)


def _finalize_body(p0_ref, p1_ref, d0_ref, d1_ref, xl_ref, xr_ref,
                   shead_ref, m_ref, bias_ref, o_ref):
    acc = p0_ref[...] + p1_ref[...]
    den = jnp.dot(d0_ref[...] + d1_ref[...], shead_ref[...],
                  preferred_element_type=jnp.float32)
    xlb = xl_ref[...]
    m = xlb + xr_ref[...]
    lr = jnp.maximum(m, 0.2 * m)
    w = jnp.exp(jnp.dot(lr, m_ref[...], preferred_element_type=jnp.float32))
    o_ref[...] = (acc + w * xlb) / (den + w + 1e-16) + bias_ref[...]


def _tc_finalize(p0, p1, d0, d1, xl, xr, shead, mblk, bias):
    B = 128
    grid = NP // B
    return pl.pallas_call(
        _finalize_body,
        grid=(grid,),
        in_specs=[
            pl.BlockSpec((B, D), lambda i: (i, 0)),
            pl.BlockSpec((B, D), lambda i: (i, 0)),
            pl.BlockSpec((B, H), lambda i: (i, 0)),
            pl.BlockSpec((B, H), lambda i: (i, 0)),
            pl.BlockSpec((B, D), lambda i: (i, 0)),
            pl.BlockSpec((B, D), lambda i: (i, 0)),
            pl.BlockSpec((H, D), lambda i: (0, 0)),
            pl.BlockSpec((D, D), lambda i: (0, 0)),
            pl.BlockSpec((1, D), lambda i: (0, 0)),
        ],
        out_specs=pl.BlockSpec((B, D), lambda i: (i, 0)),
        out_shape=jax.ShapeDtypeStruct((NP, D), jnp.float32),
    )(p0, p1, d0, d1, xl, xr, shead, mblk, bias.reshape(1, D))


def kernel(x, edge_index, Wl, bl, Wr, br, att, bias):
    n = x.shape[0]
    x_pad = jnp.pad(x, ((0, NP - n), (0, 0)))

    xl, xr = _tc_project(x_pad, Wl, bl, Wr, br)

    partial = _sc_edge_pass(edge_index[0], edge_index[1], xl, xr, att.reshape(D))

    p0 = partial[0:NP]
    d0 = partial[NP:NP + DR].reshape(NP, H)
    p1 = partial[NP + DR:2 * NP + DR]
    d1 = partial[2 * NP + DR:].reshape(NP, H)

    f = jnp.arange(D)
    shead = jnp.zeros((H, D), jnp.float32).at[f // C, f].set(1.0)
    head_eq = (f[:, None] // C) == (f[None, :] // C)
    mblk = jnp.where(head_eq, att.reshape(D)[:, None], 0.0).astype(jnp.float32)

    out = _tc_finalize(p0, p1, d0, d1, xl, xr, shead, mblk, bias)
    return out[:n]


# speedup vs baseline: 24.7367x; 1.9924x over previous
import functools

import jax
import jax.numpy as jnp
from jax import lax
from jax.experimental import pallas as pl
from jax.experimental.pallas import tpu as pltpu
from jax.experimental.pallas import tpu_sc as plsc

H = 4
C = 32
D = H * C
NC = 2
NS = 16
NW = NC * NS
CH = 80
L = 16
NP = 10240
DR = NP * H // D


def _project_body(x_ref, wl_ref, bl_ref, wr_ref, br_ref, xl_ref, xr_ref):
    xb = x_ref[...]
    xl_ref[...] = jnp.dot(xb, wl_ref[...], preferred_element_type=jnp.float32) + bl_ref[...]
    xr_ref[...] = jnp.dot(xb, wr_ref[...], preferred_element_type=jnp.float32) + br_ref[...]


def _tc_project(x, Wl, bl, Wr, br):
    n = x.shape[0]
    B = 512
    grid = n // B
    return pl.pallas_call(
        _project_body,
        grid=(grid,),
        in_specs=[
            pl.BlockSpec((B, D), lambda i: (i, 0)),
            pl.BlockSpec((D, D), lambda i: (0, 0)),
            pl.BlockSpec((1, D), lambda i: (0, 0)),
            pl.BlockSpec((D, D), lambda i: (0, 0)),
            pl.BlockSpec((1, D), lambda i: (0, 0)),
        ],
        out_specs=[
            pl.BlockSpec((B, D), lambda i: (i, 0)),
            pl.BlockSpec((B, D), lambda i: (i, 0)),
        ],
        out_shape=[
            jax.ShapeDtypeStruct((n, D), jnp.float32),
            jax.ShapeDtypeStruct((n, D), jnp.float32),
        ],
    )(x, Wl, bl.reshape(1, D), Wr, br.reshape(1, D))


def _sc_edge_pass(src, dst, xl, xr, att):
    e = src.shape[0]
    ept = e // NW
    nchunk = ept // CH
    npt = NP // NS
    nz = 40
    dpt = DR // 8

    mesh = plsc.VectorSubcoreMesh(
        core_axis_name="c", subcore_axis_name="s", num_cores=NC, num_subcores=NS)

    @functools.partial(
        pl.kernel,
        out_type=jax.ShapeDtypeStruct((NC * (NP + DR), D), jnp.float32),
        mesh=mesh,
        scratch_types=[
            pltpu.VMEM((CH,), jnp.int32),
            pltpu.VMEM((CH,), jnp.int32),
            pltpu.VMEM((CH,), jnp.int32),
            pltpu.VMEM((CH, D), jnp.float32),
            pltpu.VMEM((CH, D), jnp.float32),
            pltpu.VMEM((CH, D), jnp.float32),
            pltpu.VMEM((nz, D), jnp.float32),
            pltpu.VMEM((D,), jnp.float32),
            pltpu.VMEM_SHARED((NP + DR, D), jnp.float32),
            pltpu.SemaphoreType.DMA,
            pltpu.SemaphoreType.DMA,
        ],
        compiler_params=pltpu.CompilerParams(needs_layout_passes=False),
    )
    def edge_kernel(src_hbm, dst_hbm, xl_hbm, xr_hbm, att_hbm, out_hbm,
                    src_v, dst_v, didx, xl_rows, xr_rows, dbuf, zbuf, att_v,
                    acc_sh, sem1, sem2):
        cid = lax.axis_index("c")
        sid = lax.axis_index("s")

        pltpu.sync_copy(att_hbm, att_v)

        zero16 = jnp.zeros((L,), jnp.float32)
        lane = lax.iota(jnp.int32, L)

        def zrow(i, _):
            def zcol(j, _):
                zbuf[i, pl.ds(j * L, L)] = zero16
                return 0
            lax.fori_loop(0, D // L, zcol, 0)
            return 0
        lax.fori_loop(0, nz, zrow, 0)

        rbase = sid * npt
        for k in range(npt // nz):
            pltpu.sync_copy(zbuf, acc_sh.at[pl.ds(rbase + k * nz, nz)])

        @pl.when(sid < 8)
        def _():
            pltpu.sync_copy(zbuf, acc_sh.at[pl.ds(NP + sid * dpt, dpt)])

        def zdrow(i, _):
            def zdcol(j, _):
                dbuf[i, pl.ds(j * L, L)] = zero16
                return 0
            lax.fori_loop(0, D // L, zdcol, 0)
            return 0
        lax.fori_loop(0, CH, zdrow, 0)

        plsc.subcore_barrier()

        ebase = (cid * NS + sid) * ept

        def chunk(i, _):
            off = ebase + i * CH
            pltpu.sync_copy(src_hbm.at[pl.ds(off, CH)], src_v)
            pltpu.sync_copy(dst_hbm.at[pl.ds(off, CH)], dst_v)
            cpl = pltpu.async_copy(xl_hbm.at[src_v], xl_rows, sem1)
            cpr = pltpu.async_copy(xr_hbm.at[dst_v], xr_rows, sem2)
            cpl.wait()
            cpr.wait()

            dcols = []
            for g in range(CH // L):
                eids = g * L + lane
                dv = plsc.load_gather(dst_v, [eids])
                didx[pl.ds(g * L, L)] = lax.shift_right_logical(dv, 5) + NP
                for h in range(H):
                    U = 8

                    def accum(cb, acc):
                        for u in range(U):
                            j = cb * U + u
                            f = h * C + jnp.bitwise_and(lane + j, C - 1)
                            m = (plsc.load_gather(xl_rows, [eids, f])
                                 + plsc.load_gather(xr_rows, [eids, f]))
                            lr = jnp.maximum(m, 0.2 * m)
                            coef = plsc.load_gather(att_v, [f])
                            acc = acc + coef * lr
                        return acc
                    ev = lax.fori_loop(0, C // U, accum,
                                       jnp.zeros((L,), jnp.float32))
                    w = jnp.exp(ev)
                    col = jnp.bitwise_and(dv * 4 + h, 127)
                    plsc.store_scatter(dbuf, [eids, col], w)
                    dcols.append((eids, col))

                    def scale(cb, _):
                        for u in range(U):
                            j = cb * U + u
                            f = h * C + jnp.bitwise_and(lane + j, C - 1)
                            a = plsc.load_gather(xl_rows, [eids, f])
                            plsc.store_scatter(xl_rows, [eids, f], w * a)
                        return 0
                    lax.fori_loop(0, C // U, scale, 0)

            pltpu.sync_copy(xl_rows, acc_sh.at[dst_v], add=True)
            pltpu.sync_copy(dbuf, acc_sh.at[didx], add=True)
            for eids, col in dcols:
                plsc.store_scatter(dbuf, [eids, col], jnp.zeros((L,), jnp.float32))
            return 0

        lax.fori_loop(0, nchunk, chunk, 0)

        plsc.subcore_barrier()

        obase = cid * (NP + DR) + sid * npt
        for k in range(npt // nz):
            pltpu.sync_copy(acc_sh.at[pl.ds(rbase + k * nz, nz)],
                            out_hbm.at[pl.ds(obase + k * nz, nz)])

        @pl.when(sid < 8)
        def _():
            dbase = cid * (NP + DR) + NP + sid * dpt
            pltpu.sync_copy(acc_sh.at[pl.ds(NP + sid * dpt, dpt)],
                            out_hbm.at[pl.ds(dbase, dpt)])

    return edge_kernel(src, dst, xl, xr, att)


def _finalize_body(p0_ref, p1_ref, d0_ref, d1_ref, xl_ref, xr_ref,
                   shead_ref, m_ref, bias_ref, o_ref):
    acc = p0_ref[...] + p1_ref[...]
    den = jnp.dot(d0_ref[...] + d1_ref[...], shead_ref[...],
                  preferred_element_type=jnp.float32)
    xlb = xl_ref[...]
    m = xlb + xr_ref[...]
    lr = jnp.maximum(m, 0.2 * m)
    w = jnp.exp(jnp.dot(lr, m_ref[...], preferred_element_type=jnp.float32))
    o_ref[...] = (acc + w * xlb) / (den + w + 1e-16) + bias_ref[...]


def _tc_finalize(p0, p1, d0, d1, xl, xr, shead, mblk, bias):
    B = 128
    grid = NP // B
    return pl.pallas_call(
        _finalize_body,
        grid=(grid,),
        in_specs=[
            pl.BlockSpec((B, D), lambda i: (i, 0)),
            pl.BlockSpec((B, D), lambda i: (i, 0)),
            pl.BlockSpec((B, H), lambda i: (i, 0)),
            pl.BlockSpec((B, H), lambda i: (i, 0)),
            pl.BlockSpec((B, D), lambda i: (i, 0)),
            pl.BlockSpec((B, D), lambda i: (i, 0)),
            pl.BlockSpec((H, D), lambda i: (0, 0)),
            pl.BlockSpec((D, D), lambda i: (0, 0)),
            pl.BlockSpec((1, D), lambda i: (0, 0)),
        ],
        out_specs=pl.BlockSpec((B, D), lambda i: (i, 0)),
        out_shape=jax.ShapeDtypeStruct((NP, D), jnp.float32),
    )(p0, p1, d0, d1, xl, xr, shead, mblk, bias.reshape(1, D))


def kernel(x, edge_index, Wl, bl, Wr, br, att, bias):
    n = x.shape[0]
    x_pad = jnp.pad(x, ((0, NP - n), (0, 0)))

    xl, xr = _tc_project(x_pad, Wl, bl, Wr, br)

    partial = _sc_edge_pass(edge_index[0], edge_index[1], xl, xr, att.reshape(D))

    p0 = partial[0:NP]
    d0 = partial[NP:NP + DR].reshape(NP, H)
    p1 = partial[NP + DR:2 * NP + DR]
    d1 = partial[2 * NP + DR:].reshape(NP, H)

    f = jnp.arange(D)
    shead = jnp.zeros((H, D), jnp.float32).at[f // C, f].set(1.0)
    head_eq = (f[:, None] // C) == (f[None, :] // C)
    mblk = jnp.where(head_eq, att.reshape(D)[:, None], 0.0).astype(jnp.float32)

    out = _tc_finalize(p0, p1, d0, d1, xl, xr, shead, mblk, bias)
    return out[:n]


# speedup vs baseline: 41.4959x; 1.6775x over previous
import functools

import jax
import jax.numpy as jnp
from jax import lax
from jax.experimental import pallas as pl
from jax.experimental.pallas import tpu as pltpu
from jax.experimental.pallas import tpu_sc as plsc

H = 4
C = 32
D = H * C
NC = 2
NS = 16
NW = NC * NS
CH = 64
L = 16
NP = 10240
DR = NP * H // D


def _project_body(x_ref, wl_ref, bl_ref, wr_ref, br_ref, xl_ref, xr_ref):
    xb = x_ref[...]
    xl_ref[...] = jnp.dot(xb, wl_ref[...], preferred_element_type=jnp.float32) + bl_ref[...]
    xr_ref[...] = jnp.dot(xb, wr_ref[...], preferred_element_type=jnp.float32) + br_ref[...]


def _tc_project(x, Wl, bl, Wr, br):
    n = x.shape[0]
    B = 512
    grid = n // B
    return pl.pallas_call(
        _project_body,
        grid=(grid,),
        in_specs=[
            pl.BlockSpec((B, D), lambda i: (i, 0)),
            pl.BlockSpec((D, D), lambda i: (0, 0)),
            pl.BlockSpec((1, D), lambda i: (0, 0)),
            pl.BlockSpec((D, D), lambda i: (0, 0)),
            pl.BlockSpec((1, D), lambda i: (0, 0)),
        ],
        out_specs=[
            pl.BlockSpec((B, D), lambda i: (i, 0)),
            pl.BlockSpec((B, D), lambda i: (i, 0)),
        ],
        out_shape=[
            jax.ShapeDtypeStruct((n, D), jnp.float32),
            jax.ShapeDtypeStruct((n, D), jnp.float32),
        ],
    )(x, Wl, bl.reshape(1, D), Wr, br.reshape(1, D))


def _sc_edge_pass(src, dst, xl, xr, att):
    npt = NP // NS
    nch = src.shape[0] // (NW * CH)
    NPAIR = nch // 2

    mesh = plsc.VectorSubcoreMesh(
        core_axis_name="c", subcore_axis_name="s", num_cores=NC, num_subcores=NS)

    @functools.partial(
        pl.kernel,
        out_type=jax.ShapeDtypeStruct((NC * (NP + DR), D), jnp.float32),
        mesh=mesh,
        scratch_types=[
            pltpu.VMEM((CH,), jnp.int32),
            pltpu.VMEM((CH,), jnp.int32),
            pltpu.VMEM((CH,), jnp.int32),
            pltpu.VMEM((CH,), jnp.int32),
            pltpu.VMEM((CH,), jnp.int32),
            pltpu.VMEM((CH, D), jnp.float32),
            pltpu.VMEM((CH, D), jnp.float32),
            pltpu.VMEM((CH, D), jnp.float32),
            pltpu.VMEM((CH, D), jnp.float32),
            pltpu.VMEM((CH, D), jnp.float32),
            pltpu.VMEM((D,), jnp.float32),
            pltpu.VMEM_SHARED((NP + DR, D), jnp.float32),
            pltpu.SemaphoreType.DMA,
            pltpu.SemaphoreType.DMA,
            pltpu.SemaphoreType.DMA,
            pltpu.SemaphoreType.DMA,
        ],
        compiler_params=pltpu.CompilerParams(needs_layout_passes=False),
    )
    def edge_kernel(src_hbm, dst_hbm, xl_hbm, xr_hbm, att_hbm, out_hbm,
                    src_va, dst_va, src_vb, dst_vb, didx,
                    xl_a, xr_a, xl_b, xr_b, dbuf, att_v,
                    acc_sh, sla, sra, slb, srb):
        cid = lax.axis_index("c")
        sid = lax.axis_index("s")
        tg = cid * NS + sid
        ebase = CH * nch * tg

        pltpu.sync_copy(att_hbm, att_v)

        zero16 = jnp.zeros((L,), jnp.float32)
        lane = lax.iota(jnp.int32, L)

        def zdrow(i, _):
            def zdcol(j, _):
                dbuf[i, pl.ds(j * L, L)] = zero16
                return 0
            lax.fori_loop(0, D // L, zdcol, 0)
            return 0
        lax.fori_loop(0, CH, zdrow, 0)

        rbase = sid * npt
        for k in range(npt // CH):
            pltpu.sync_copy(dbuf, acc_sh.at[pl.ds(rbase + k * CH, CH)])

        @pl.when(sid < DR // CH)
        def _():
            pltpu.sync_copy(dbuf, acc_sh.at[pl.ds(NP + sid * CH, CH)])

        plsc.subcore_barrier()

        def fetch(c, src_v, dst_v, xl_rows, xr_rows, sl, sr):
            off = ebase + c * CH
            pltpu.sync_copy(src_hbm.at[pl.ds(off, CH)], src_v)
            pltpu.sync_copy(dst_hbm.at[pl.ds(off, CH)], dst_v)
            pltpu.async_copy(xl_hbm.at[src_v], xl_rows, sl)
            pltpu.async_copy(xr_hbm.at[dst_v], xr_rows, sr)

        def wait_pair(src_v, dst_v, xl_rows, xr_rows, sl, sr):
            pltpu.make_async_copy(xl_hbm.at[src_v], xl_rows, sl).wait()
            pltpu.make_async_copy(xr_hbm.at[dst_v], xr_rows, sr).wait()

        def compute_chunk(src_v, dst_v, xl_rows, xr_rows):
            def grp(g, _):
                eids = g * L + lane
                dv = plsc.load_gather(dst_v, [eids])
                didx[pl.ds(g * L, L)] = lax.shift_right_logical(dv, 5) + NP
                for h in range(H):
                    U = 8

                    def accum(cb, acc):
                        for u in range(U):
                            j = cb * U + u
                            f = h * C + jnp.bitwise_and(lane + j, C - 1)
                            m = (plsc.load_gather(xl_rows, [eids, f])
                                 + plsc.load_gather(xr_rows, [eids, f]))
                            lr = jnp.maximum(m, 0.2 * m)
                            coef = plsc.load_gather(att_v, [f])
                            acc = acc + coef * lr
                        return acc
                    ev = lax.fori_loop(0, C // U, accum,
                                       jnp.zeros((L,), jnp.float32))
                    w = jnp.exp(ev)
                    col = jnp.bitwise_and(dv * 4 + h, 127)
                    plsc.store_scatter(dbuf, [eids, col], w)

                    def scale(cb, _):
                        for u in range(U):
                            j = cb * U + u
                            f = h * C + jnp.bitwise_and(lane + j, C - 1)
                            a = plsc.load_gather(xl_rows, [eids, f])
                            plsc.store_scatter(xl_rows, [eids, f], w * a)
                        return 0
                    lax.fori_loop(0, C // U, scale, 0)
                return 0
            lax.fori_loop(0, CH // L, grp, 0)

            pltpu.sync_copy(xl_rows, acc_sh.at[dst_v], add=True)
            pltpu.sync_copy(dbuf, acc_sh.at[didx], add=True)

            def rz(g, _):
                eids = g * L + lane
                dv = plsc.load_gather(dst_v, [eids])
                for h in range(H):
                    col = jnp.bitwise_and(dv * 4 + h, 127)
                    plsc.store_scatter(dbuf, [eids, col],
                                       jnp.zeros((L,), jnp.float32))
                return 0
            lax.fori_loop(0, CH // L, rz, 0)

        fetch(0, src_va, dst_va, xl_a, xr_a, sla, sra)
        fetch(1, src_vb, dst_vb, xl_b, xr_b, slb, srb)

        def pair(j, _):
            wait_pair(src_va, dst_va, xl_a, xr_a, sla, sra)
            compute_chunk(src_va, dst_va, xl_a, xr_a)

            @pl.when(2 * j + 2 < nch)
            def _():
                fetch(2 * j + 2, src_va, dst_va, xl_a, xr_a, sla, sra)

            wait_pair(src_vb, dst_vb, xl_b, xr_b, slb, srb)
            compute_chunk(src_vb, dst_vb, xl_b, xr_b)

            @pl.when(2 * j + 3 < nch)
            def _():
                fetch(2 * j + 3, src_vb, dst_vb, xl_b, xr_b, slb, srb)
            return 0

        lax.fori_loop(0, NPAIR, pair, 0)

        plsc.subcore_barrier()

        obase = cid * (NP + DR) + rbase
        for k in range(npt // CH):
            pltpu.sync_copy(acc_sh.at[pl.ds(rbase + k * CH, CH)],
                            out_hbm.at[pl.ds(obase + k * CH, CH)])

        @pl.when(sid < DR // CH)
        def _():
            dbase = cid * (NP + DR) + NP + sid * CH
            pltpu.sync_copy(acc_sh.at[pl.ds(NP + sid * CH, CH)],
                            out_hbm.at[pl.ds(dbase, CH)])

    return edge_kernel(src, dst, xl, xr, att)


def _finalize_body(p0_ref, p1_ref, d0_ref, d1_ref, xl_ref, xr_ref,
                   shead_ref, m_ref, bias_ref, o_ref):
    acc = p0_ref[...] + p1_ref[...]
    den = jnp.dot(d0_ref[...] + d1_ref[...], shead_ref[...],
                  preferred_element_type=jnp.float32)
    xlb = xl_ref[...]
    m = xlb + xr_ref[...]
    lr = jnp.maximum(m, 0.2 * m)
    w = jnp.exp(jnp.dot(lr, m_ref[...], preferred_element_type=jnp.float32))
    o_ref[...] = (acc + w * xlb) / (den + w + 1e-16) + bias_ref[...]


def _tc_finalize(p0, p1, d0, d1, xl, xr, shead, mblk, bias):
    B = 128
    grid = NP // B
    return pl.pallas_call(
        _finalize_body,
        grid=(grid,),
        in_specs=[
            pl.BlockSpec((B, D), lambda i: (i, 0)),
            pl.BlockSpec((B, D), lambda i: (i, 0)),
            pl.BlockSpec((B, H), lambda i: (i, 0)),
            pl.BlockSpec((B, H), lambda i: (i, 0)),
            pl.BlockSpec((B, D), lambda i: (i, 0)),
            pl.BlockSpec((B, D), lambda i: (i, 0)),
            pl.BlockSpec((H, D), lambda i: (0, 0)),
            pl.BlockSpec((D, D), lambda i: (0, 0)),
            pl.BlockSpec((1, D), lambda i: (0, 0)),
        ],
        out_specs=pl.BlockSpec((B, D), lambda i: (i, 0)),
        out_shape=jax.ShapeDtypeStruct((NP, D), jnp.float32),
    )(p0, p1, d0, d1, xl, xr, shead, mblk, bias.reshape(1, D))


def kernel(x, edge_index, Wl, bl, Wr, br, att, bias):
    n = x.shape[0]
    x_pad = jnp.pad(x, ((0, NP - n), (0, 0)))

    xl, xr = _tc_project(x_pad, Wl, bl, Wr, br)

    e = edge_index.shape[1]
    ep = NW * CH * (2 * -(-e // (NW * CH * 2)))
    src_p = jnp.pad(edge_index[0], (0, ep - e), constant_values=NP - 1)
    dst_p = jnp.pad(edge_index[1], (0, ep - e), constant_values=NP - 1)

    partial = _sc_edge_pass(src_p, dst_p, xl, xr, att.reshape(D))

    p0 = partial[0:NP]
    d0 = partial[NP:NP + DR].reshape(NP, H)
    p1 = partial[NP + DR:2 * NP + DR]
    d1 = partial[2 * NP + DR:].reshape(NP, H)

    f = jnp.arange(D)
    shead = jnp.zeros((H, D), jnp.float32).at[f // C, f].set(1.0)
    head_eq = (f[:, None] // C) == (f[None, :] // C)
    mblk = jnp.where(head_eq, att.reshape(D)[:, None], 0.0).astype(jnp.float32)

    out = _tc_finalize(p0, p1, d0, d1, xl, xr, shead, mblk, bias)
    return out[:n]
